# Initial kernel scaffold; baseline (speedup 1.0000x reference)
import jax
import jax.numpy as jnp
from jax.experimental import pallas as pl


def kernel(x, edge_index, W1, b1, W2, b2, W3, b3):
    raise NotImplementedError("write your pallas kernel here")


# speedup vs baseline: 5.8726x; 5.8726x over previous
import functools

import jax
import jax.numpy as jnp
from jax import lax
from jax.experimental import pallas as pl
from jax.experimental.pallas import tpu as pltpu
from jax.experimental.pallas import tpu_sc as plsc

N = 10000
E = 160000
DIN = 256
DH = 512
NC = 2
NS = 16
NCHUNK = 4
CW = 128
RPT = N // NS

DEG_B = 40
DEG_NB = (E // (NC * NS)) // DEG_B
SC_B = 80
SC_NB = (E // NS) // SC_B

MB = 2000
_MESH = plsc.VectorSubcoreMesh(core_axis_name="c", subcore_axis_name="s",
                               num_cores=NC, num_subcores=NS)


def _zero_rows(ref, nrows, ncols):
    z = jnp.zeros((16,), jnp.float32)

    def body(i, _):
        for j in range(ncols // 16):
            ref[i, j * 16:(j + 1) * 16] = z
        return 0

    lax.fori_loop(0, nrows, body, 0)


def _deg_body(dst_hbm, out_hbm, dstv, onesv, zerov, degsh, sem):
    cid = lax.axis_index("c")
    sid = lax.axis_index("s")
    wid = cid * NS + sid
    e0 = jnp.where(lax.iota(jnp.int32, 16) == 0,
                   jnp.float32(1.0), jnp.float32(0.0))

    def fill(i, _):
        onesv[i, :] = e0
        return 0

    lax.fori_loop(0, DEG_B, fill, 0)
    _zero_rows(zerov, RPT, 16)
    pltpu.sync_copy(dst_hbm.at[wid], dstv)
    pltpu.async_copy(zerov, degsh.at[pl.ds(sid * RPT, RPT)], sem).wait()
    plsc.subcore_barrier()

    def body(b, _):
        pltpu.async_copy(onesv, degsh.at[dstv.at[b]], sem, add=True).wait()
        return 0

    lax.fori_loop(0, DEG_NB, body, 0)
    plsc.subcore_barrier()
    pltpu.async_copy(degsh.at[pl.ds(sid * RPT, RPT)], zerov, sem).wait()
    pltpu.sync_copy(zerov, out_hbm.at[cid, sid])


_deg_call = pl.kernel(
    _deg_body,
    out_type=jax.ShapeDtypeStruct((NC, NS, RPT, 16), jnp.float32),
    mesh=_MESH,
    scratch_types=[
        pltpu.VMEM((DEG_NB, DEG_B), jnp.int32),
        pltpu.VMEM((DEG_B, 16), jnp.float32),
        pltpu.VMEM((RPT, 16), jnp.float32),
        pltpu.VMEM_SHARED((N, 16), jnp.float32),
        pltpu.SemaphoreType.DMA,
    ],
    compiler_params=pltpu.CompilerParams(use_tc_tiling_on_sc=False),
)


def _scatter_body(src_hbm, dstw_hbm, u_hbm, out_hbm,
                  srcb, dstb, zerov, rowsv, aggsh, sem):
    cid = lax.axis_index("c")
    sid = lax.axis_index("s")
    ZR = RPT // 25
    for j in range(NCHUNK // NC):
        cc = cid * (NCHUNK // NC) + j
        off = cc * N
        _zero_rows(zerov, ZR, CW)
        for k in range(25):
            pltpu.async_copy(zerov,
                             aggsh.at[pl.ds(sid * RPT + k * ZR, ZR)],
                             sem).wait()
        plsc.subcore_barrier()

        def body(b, _):
            pltpu.sync_copy(dstw_hbm.at[sid, b], dstb.at[0])
            pltpu.sync_copy(src_hbm.at[sid, b], srcb.at[0])
            for jj in range(SC_B // 16):
                srcb[0, jj * 16:(jj + 1) * 16] = (
                    srcb[0, jj * 16:(jj + 1) * 16] + off)
            pltpu.async_copy(u_hbm.at[srcb.at[0]], rowsv, sem).wait()
            pltpu.async_copy(rowsv, aggsh.at[dstb.at[0]], sem,
                             add=True).wait()
            return 0

        lax.fori_loop(0, SC_NB, body, 0)
        plsc.subcore_barrier()
        for k in range(25):
            pltpu.async_copy(aggsh.at[pl.ds(sid * RPT + k * ZR, ZR)],
                             zerov, sem).wait()
            pltpu.sync_copy(zerov, out_hbm.at[cc, sid, pl.ds(k * ZR, ZR)])


_scatter_call = pl.kernel(
    _scatter_body,
    out_type=jax.ShapeDtypeStruct((NCHUNK, NS, RPT, CW), jnp.float32),
    mesh=_MESH,
    scratch_types=[
        pltpu.VMEM((1, SC_B), jnp.int32),
        pltpu.VMEM((1, SC_B), jnp.int32),
        pltpu.VMEM((RPT // 25, CW), jnp.float32),
        pltpu.VMEM((SC_B, CW), jnp.float32),
        pltpu.VMEM_SHARED((N, CW), jnp.float32),
        pltpu.SemaphoreType.DMA,
    ],
    compiler_params=pltpu.CompilerParams(use_tc_tiling_on_sc=False),
)


def _t1_body(x_ref, w_ref, d0_ref, d1_ref, u_ref, dinv_ref):
    deg = d0_ref[...] + d1_ref[...] + 1.0
    dinv = lax.rsqrt(deg)
    u = jnp.dot(x_ref[...], w_ref[...],
                preferred_element_type=jnp.float32) * dinv
    dinv_ref[...] = dinv
    for c in range(NCHUNK):
        u_ref[c] = u[:, c * CW:(c + 1) * CW]


def _tmid_body(s_ref, u_ref, dinv_ref, b_ref, w_ref, o_ref):
    dinv = dinv_ref[...]
    h = jnp.concatenate([s_ref[c] + u_ref[c] for c in range(NCHUNK)], axis=1)
    h = jnp.maximum(h * dinv + b_ref[...], 0.0)
    o = jnp.dot(h, w_ref[...], preferred_element_type=jnp.float32) * dinv
    for c in range(NCHUNK):
        o_ref[c] = o[:, c * CW:(c + 1) * CW]


def _tfin_body(s_ref, u_ref, dinv_ref, b_ref, z_ref):
    dinv = dinv_ref[...]
    h = jnp.concatenate([s_ref[c] + u_ref[c] for c in range(NCHUNK)], axis=1)
    z_ref[...] = h * dinv + b_ref[...]


def _chunked_spec():
    return pl.BlockSpec((NCHUNK, MB, CW), lambda m: (0, m, 0))


def _col_spec():
    return pl.BlockSpec((MB, 1), lambda m: (m, 0))


def _t1(x, w1, d0, d1):
    return pl.pallas_call(
        _t1_body,
        grid=(N // MB,),
        in_specs=[
            pl.BlockSpec((MB, DIN), lambda m: (m, 0)),
            pl.BlockSpec((DIN, DH), lambda m: (0, 0)),
            _col_spec(),
            _col_spec(),
        ],
        out_specs=[_chunked_spec(), _col_spec()],
        out_shape=[
            jax.ShapeDtypeStruct((NCHUNK, N, CW), jnp.float32),
            jax.ShapeDtypeStruct((N, 1), jnp.float32),
        ],
    )(x, w1, d0, d1)


def _tmid(s, u, dinv, b, w):
    return pl.pallas_call(
        _tmid_body,
        grid=(N // MB,),
        in_specs=[
            _chunked_spec(),
            _chunked_spec(),
            _col_spec(),
            pl.BlockSpec((1, DH), lambda m: (0, 0)),
            pl.BlockSpec((DH, DH), lambda m: (0, 0)),
        ],
        out_specs=_chunked_spec(),
        out_shape=jax.ShapeDtypeStruct((NCHUNK, N, CW), jnp.float32),
    )(s, u, dinv, b, w)


def _tfin(s, u, dinv, b):
    return pl.pallas_call(
        _tfin_body,
        grid=(N // MB,),
        in_specs=[
            _chunked_spec(),
            _chunked_spec(),
            _col_spec(),
            pl.BlockSpec((1, DH), lambda m: (0, 0)),
        ],
        out_specs=pl.BlockSpec((MB, DH), lambda m: (m, 0)),
        out_shape=jax.ShapeDtypeStruct((N, DH), jnp.float32),
    )(s, u, dinv, b)


def kernel(x, edge_index, W1, b1, W2, b2, W3, b3):
    src = edge_index[0]
    dst = edge_index[1]
    dst_deg = dst.reshape(NC * NS, DEG_NB, DEG_B)
    srcw = src.reshape(NS, SC_NB, SC_B)
    dstw = dst.reshape(NS, SC_NB, SC_B)

    degp = _deg_call(dst_deg).reshape(NC, N, 16)
    d0 = degp[0, :, :1]
    d1 = degp[1, :, :1]

    def scat(u):
        return _scatter_call(srcw, dstw,
                             u.reshape(NCHUNK * N, CW)).reshape(NCHUNK, N, CW)

    u1, dinv = _t1(x, W1, d0, d1)
    s1 = scat(u1)
    u2 = _tmid(s1, u1, dinv, b1.reshape(1, DH), W2)
    s2 = scat(u2)
    u3 = _tmid(s2, u2, dinv, b2.reshape(1, DH), W3)
    s3 = scat(u3)
    return _tfin(s3, u3, dinv, b3.reshape(1, DH))
